# baseline (speedup 1.0000x reference)
import functools

import jax
import jax.numpy as jnp
from jax import lax
from jax.experimental import pallas as pl
from jax.experimental.pallas import tpu as pltpu
from jax.experimental.pallas import tpu_sc as plsc

N = 50000
E = 800000
D = 64
H = 32
NC = 2
NS = 16
NP = 50176
SLD = NP // NS
SLX = NP // (NC * NS)
WD = 1000
WS = 400
SLN = N // NS
F32 = jnp.float32
I32 = jnp.int32


@functools.cache
def _mesh():
    return plsc.VectorSubcoreMesh(
        core_axis_name="c", subcore_axis_name="s", num_cores=NC, num_subcores=NS
    )


_SC_PARAMS = pltpu.CompilerParams(
    use_tc_tiling_on_sc=False, needs_layout_passes=False
)


@functools.cache
def _sc_degree_kernel():
    return pl.kernel(
        _sc_degree_body,
        out_type=(
            jax.ShapeDtypeStruct((NP,), F32),
            jax.ShapeDtypeStruct((NP, H), F32),
            jax.ShapeDtypeStruct((NP, H), F32),
        ),
        mesh=_mesh(),
        scratch_types=[
            pltpu.VMEM((WD,), I32),
            pltpu.VMEM((WD,), F32),
            pltpu.VMEM((SLD,), F32),
            pltpu.VMEM((SLX,), F32),
            pltpu.VMEM((SLX,), F32),
            pltpu.VMEM((WS, H), F32),
            pltpu.VMEM_SHARED((NP,), F32),
        ],
        compiler_params=_SC_PARAMS,
    )


_XCHUNKS = [(o, min(WS, SLX - o)) for o in range(0, SLX, WS)]


def _rsqrt16(v):
    i = plsc.bitcast(v, I32)
    y = plsc.bitcast(1597463007 - lax.shift_right_logical(i, 1), F32)
    for _ in range(4):
        t = v * y
        t = t * y
        y = y * (1.5 - 0.5 * t)
    return y


def _sc_degree_body(dst_hbm, ones_hbm, zdeg_hbm, deg_hbm, d2x_hbm, sqx_hbm,
                    didx, ones_v, stage, dsl, sqv, xbuf, acc):
    c = lax.axis_index("c")
    s = lax.axis_index("s")
    pltpu.sync_copy(zdeg_hbm, stage)
    pltpu.sync_copy(stage, acc.at[pl.ds(s * SLD, SLD)])
    pltpu.sync_copy(ones_hbm, ones_v)
    plsc.subcore_barrier()

    per_s = E // NS
    base = s * per_s

    @pl.loop(0, per_s // WD)
    def _(i):
        pltpu.sync_copy(dst_hbm.at[pl.ds(base + i * WD, WD)], didx)
        pltpu.sync_copy(ones_v, acc.at[didx], add=True)

    plsc.subcore_barrier()

    @pl.when(c == 0)
    def _():
        pltpu.sync_copy(acc.at[pl.ds(s * SLD, SLD)], stage)
        pltpu.sync_copy(stage, deg_hbm.at[pl.ds(s * SLD, SLD)])

    xbase = (c * NS + s) * SLX
    pltpu.sync_copy(acc.at[pl.ds(xbase, SLX)], dsl)

    @pl.loop(0, SLX // 16)
    def _(i):
        sl16 = pl.ds(i * 16, 16)
        v = dsl[sl16]
        sqv[sl16] = v * _rsqrt16(v)
        dsl[sl16] = jnp.where(v > 0, 1.0 / v, 0.0)

    for vals, dest in ((dsl, d2x_hbm), (sqv, sqx_hbm)):
        for off, sz in _XCHUNKS:
            @pl.loop(0, sz)
            def _(r, _off=off, _vals=vals):
                dvec = plsc.load_gather(_vals, [jnp.full((16,), _off + r, I32)])
                xbuf[r, pl.ds(0, 16)] = dvec
                xbuf[r, pl.ds(16, 16)] = dvec
            pltpu.sync_copy(xbuf.at[pl.ds(0, sz)],
                            dest.at[pl.ds(xbase + off, sz)])


@functools.cache
def _sc_scatter_kernel(mode):
    if mode == "mid":
        outs = (jax.ShapeDtypeStruct((NC, N, H), F32),
                jax.ShapeDtypeStruct((NC, N, H), F32))
    else:
        outs = jax.ShapeDtypeStruct((NC, N, H), F32)
    return pl.kernel(
        functools.partial(_sc_scatter_body, mode),
        out_type=outs,
        mesh=_mesh(),
        scratch_types=[
            pltpu.VMEM((WS,), I32),
            pltpu.VMEM((WS,), I32),
            pltpu.VMEM((WS,), I32),
            pltpu.VMEM((WS,), I32),
            pltpu.VMEM((WS, H), F32),
            pltpu.VMEM((WS, H), F32),
            pltpu.VMEM_SHARED((N, H), F32),
            pltpu.SemaphoreType.DMA,
            pltpu.SemaphoreType.DMA,
            pltpu.SemaphoreType.DMA,
            pltpu.SemaphoreType.DMA,
        ],
        compiler_params=_SC_PARAMS,
    )


_CHUNKS = [(o, min(WS, SLN - o)) for o in range(0, SLN, WS)]


def _sc_scatter_body(mode, *refs):
    if mode == "first":
        (z_hbm, src_hbm, dst_hbm, zrow_hbm, d2x_hbm, out_hbm,
         sidx0, sidx1, didx0, didx1, rows0, rows1, acc, g0, g1, t0, t1) = refs
        zs_hbm = sqx_hbm = zsum_hbm = None
    elif mode == "mid":
        (z_hbm, src_hbm, dst_hbm, zrow_hbm, d2x_hbm, out_hbm, zsum_hbm,
         sidx0, sidx1, didx0, didx1, rows0, rows1, acc, g0, g1, t0, t1) = refs
        zs_hbm = z_hbm
        sqx_hbm = None
    else:
        (z_hbm, src_hbm, dst_hbm, zrow_hbm, d2x_hbm, sqx_hbm, zs_hbm, out_hbm,
         sidx0, sidx1, didx0, didx1, rows0, rows1, acc, g0, g1, t0, t1) = refs
        zsum_hbm = None
    c = lax.axis_index("c")
    s = lax.axis_index("s")

    pltpu.sync_copy(zrow_hbm, rows0)
    for off, sz in _CHUNKS:
        pltpu.sync_copy(rows0.at[pl.ds(0, sz)],
                        acc.at[pl.ds(s * SLN + off, sz)])
    plsc.subcore_barrier()

    per_s = E // NS
    base = s * per_s

    z_c = z_hbm.at[c]
    npairs = per_s // (2 * WS)

    @pl.loop(0, npairs)
    def _(i):
        off0 = base + (2 * i) * WS
        off1 = off0 + WS
        pltpu.sync_copy(src_hbm.at[pl.ds(off0, WS)], sidx0)
        pltpu.sync_copy(dst_hbm.at[pl.ds(off0, WS)], didx0)
        gd0 = pltpu.async_copy(z_c.at[sidx0], rows0, g0)
        pltpu.sync_copy(src_hbm.at[pl.ds(off1, WS)], sidx1)
        pltpu.sync_copy(dst_hbm.at[pl.ds(off1, WS)], didx1)
        gd1 = pltpu.async_copy(z_c.at[sidx1], rows1, g1)
        gd0.wait()
        sd0 = pltpu.async_copy(rows0, acc.at[didx0], t0, add=True)
        gd1.wait()
        sd1 = pltpu.async_copy(rows1, acc.at[didx1], t1, add=True)
        sd0.wait()
        sd1.wait()

    offt = base + 2 * npairs * WS
    pltpu.sync_copy(src_hbm.at[pl.ds(offt, WS)], sidx0)
    pltpu.sync_copy(dst_hbm.at[pl.ds(offt, WS)], didx0)
    pltpu.sync_copy(z_c.at[sidx0], rows0)
    pltpu.sync_copy(rows0, acc.at[didx0], add=True)

    plsc.subcore_barrier()
    out_c = out_hbm.at[c]

    def _ewise(op, sz):
        @pl.loop(0, sz)
        def _(r):
            for hh in (0, 16):
                a = rows0[r, pl.ds(hh, 16)]
                b = rows1[r, pl.ds(hh, 16)]
                rows0[r, pl.ds(hh, 16)] = a + b if op == "add" else a * b

    for off, sz in _CHUNKS:
        sl = pl.ds(s * SLN + off, sz)
        pltpu.sync_copy(acc.at[sl], rows0.at[pl.ds(0, sz)])
        pltpu.sync_copy(d2x_hbm.at[sl], rows1.at[pl.ds(0, sz)])
        _ewise("mul", sz)
        if mode == "first":
            pltpu.sync_copy(rows0.at[pl.ds(0, sz)], out_c.at[sl])
        elif mode == "mid":
            pltpu.sync_copy(rows0.at[pl.ds(0, sz)], out_c.at[sl])
            pltpu.sync_copy(zs_hbm.at[c].at[sl], rows1.at[pl.ds(0, sz)])
            _ewise("add", sz)
            pltpu.sync_copy(rows0.at[pl.ds(0, sz)], zsum_hbm.at[c].at[sl])
        else:
            pltpu.sync_copy(zs_hbm.at[c].at[sl], rows1.at[pl.ds(0, sz)])
            _ewise("add", sz)
            pltpu.sync_copy(sqx_hbm.at[sl], rows1.at[pl.ds(0, sz)])
            _ewise("mul", sz)
            pltpu.sync_copy(rows0.at[pl.ds(0, sz)], out_c.at[sl])


def _dis_body(p_ref, dis_ref):
    deg = p_ref[...]
    dis_ref[...] = jnp.where(deg > 0, lax.rsqrt(deg), 0.0)


def _scale1_body(d_ref, x_ref, o_ref):
    d = d_ref[...]
    o_ref[0, :, :] = d * x_ref[:, :H]
    o_ref[1, :, :] = d * x_ref[:, H:]


def _final_body(e_ref, w_ref, o_ref):
    w = w_ref[...]
    cat = jnp.concatenate([w[0], w[1]], axis=1)
    o_ref[...] = 0.25 * (e_ref[...] + cat)


_BR = 5000
_COL = pl.BlockSpec((_BR, 1), lambda i: (i, 0))
_MAT = pl.BlockSpec((_BR, D), lambda i: (i, 0))
_HLF = pl.BlockSpec((NC, _BR, H), lambda i: (0, i, 0))

_tc_dis = pl.pallas_call(
    _dis_body,
    in_specs=[pl.BlockSpec((1, NP), lambda: (0, 0))],
    out_specs=pl.BlockSpec((1, NP), lambda: (0, 0)),
    out_shape=jax.ShapeDtypeStruct((1, NP), F32),
)
_tc_scale1 = pl.pallas_call(
    _scale1_body,
    grid=(N // _BR,),
    in_specs=[_COL, _MAT],
    out_specs=_HLF,
    out_shape=jax.ShapeDtypeStruct((NC, N, H), F32),
)
_tc_final = pl.pallas_call(
    _final_body,
    grid=(N // _BR,),
    in_specs=[_MAT, _HLF],
    out_specs=_MAT,
    out_shape=jax.ShapeDtypeStruct((N, D), F32),
)


def kernel(edge_index, emb):
    src = edge_index[0].astype(I32)
    dst = edge_index[1].astype(I32)

    ones_w = jnp.ones((WD,), F32)
    zdeg = jnp.zeros((SLD,), F32)
    zrow = jnp.zeros((WS, H), F32)

    deg, d2x, sqx = _sc_degree_kernel()(dst, ones_w, zdeg)
    dis = _tc_dis(deg.reshape(1, NP))
    dis_col = dis[0, :N].reshape(N, 1)

    z0 = _tc_scale1(dis_col, emb)
    z1 = _sc_scatter_kernel("first")(z0, src, dst, zrow, d2x)
    z2, zs2 = _sc_scatter_kernel("mid")(z1, src, dst, zrow, d2x)
    w3 = _sc_scatter_kernel("last")(z2, src, dst, zrow, d2x, sqx, zs2)

    return _tc_final(emb, w3)


# speedup vs baseline: 1.0314x; 1.0314x over previous
import functools

import jax
import jax.numpy as jnp
from jax import lax
from jax.experimental import pallas as pl
from jax.experimental.pallas import tpu as pltpu
from jax.experimental.pallas import tpu_sc as plsc

N = 50000
E = 800000
D = 64
H = 32
NC = 2
NS = 16
NP = 50176
SLD = NP // NS
SLX = NP // (NC * NS)
WD = 1000
WS = 400
SLN = N // NS
F32 = jnp.float32
I32 = jnp.int32


@functools.cache
def _mesh():
    return plsc.VectorSubcoreMesh(
        core_axis_name="c", subcore_axis_name="s", num_cores=NC, num_subcores=NS
    )


_SC_PARAMS = pltpu.CompilerParams(
    use_tc_tiling_on_sc=False, needs_layout_passes=False
)


@functools.cache
def _sc_degree_kernel():
    return pl.kernel(
        _sc_degree_body,
        out_type=(
            jax.ShapeDtypeStruct((NP,), F32),
            jax.ShapeDtypeStruct((NP, H), F32),
        ),
        mesh=_mesh(),
        scratch_types=[
            pltpu.VMEM((WD,), I32),
            pltpu.VMEM((WD,), F32),
            pltpu.VMEM((SLD,), F32),
            pltpu.VMEM((SLX,), F32),
            pltpu.VMEM((WS, H), F32),
            pltpu.VMEM_SHARED((NP,), F32),
        ],
        compiler_params=_SC_PARAMS,
    )


_XCHUNKS = [(o, min(WS, SLX - o)) for o in range(0, SLX, WS)]


def _sc_degree_body(dst_hbm, ones_hbm, zdeg_hbm, deg_hbm, d2x_hbm,
                    didx, ones_v, stage, dsl, xbuf, acc):
    c = lax.axis_index("c")
    s = lax.axis_index("s")
    pltpu.sync_copy(zdeg_hbm, stage)
    pltpu.sync_copy(stage, acc.at[pl.ds(s * SLD, SLD)])
    pltpu.sync_copy(ones_hbm, ones_v)
    plsc.subcore_barrier()

    per_s = E // NS
    base = s * per_s

    @pl.loop(0, per_s // WD)
    def _(i):
        pltpu.sync_copy(dst_hbm.at[pl.ds(base + i * WD, WD)], didx)
        pltpu.sync_copy(ones_v, acc.at[didx], add=True)

    plsc.subcore_barrier()

    @pl.when(c == 0)
    def _():
        pltpu.sync_copy(acc.at[pl.ds(s * SLD, SLD)], stage)
        pltpu.sync_copy(stage, deg_hbm.at[pl.ds(s * SLD, SLD)])

    xbase = (c * NS + s) * SLX
    pltpu.sync_copy(acc.at[pl.ds(xbase, SLX)], dsl)

    @pl.loop(0, SLX // 16)
    def _(i):
        v = dsl[pl.ds(i * 16, 16)]
        dsl[pl.ds(i * 16, 16)] = jnp.where(v > 0, 1.0 / v, 0.0)

    for off, sz in _XCHUNKS:
        @pl.loop(0, sz)
        def _(r, _off=off):
            dvec = plsc.load_gather(dsl, [jnp.full((16,), _off + r, I32)])
            xbuf[r, pl.ds(0, 16)] = dvec
            xbuf[r, pl.ds(16, 16)] = dvec
        pltpu.sync_copy(xbuf.at[pl.ds(0, sz)],
                        d2x_hbm.at[pl.ds(xbase + off, sz)])


@functools.cache
def _sc_scatter_kernel():
    return pl.kernel(
        _sc_scatter_body,
        out_type=jax.ShapeDtypeStruct((NC, N, H), F32),
        mesh=_mesh(),
        scratch_types=[
            pltpu.VMEM((WS,), I32),
            pltpu.VMEM((WS,), I32),
            pltpu.VMEM((WS,), I32),
            pltpu.VMEM((WS,), I32),
            pltpu.VMEM((WS, H), F32),
            pltpu.VMEM((WS, H), F32),
            pltpu.VMEM_SHARED((N, H), F32),
            pltpu.SemaphoreType.DMA,
            pltpu.SemaphoreType.DMA,
            pltpu.SemaphoreType.DMA,
            pltpu.SemaphoreType.DMA,
        ],
        compiler_params=_SC_PARAMS,
    )


_CHUNKS = [(o, min(WS, SLN - o)) for o in range(0, SLN, WS)]


def _sc_scatter_body(z_hbm, src_hbm, dst_hbm, zrow_hbm, d2x_hbm, out_hbm,
                     sidx0, sidx1, didx0, didx1, rows0, rows1, acc,
                     g0, g1, t0, t1):
    c = lax.axis_index("c")
    s = lax.axis_index("s")

    pltpu.sync_copy(zrow_hbm, rows0)
    for off, sz in _CHUNKS:
        pltpu.sync_copy(rows0.at[pl.ds(0, sz)],
                        acc.at[pl.ds(s * SLN + off, sz)])
    plsc.subcore_barrier()

    per_s = E // NS
    base = s * per_s

    z_c = z_hbm.at[c]
    npairs = per_s // (2 * WS)

    @pl.loop(0, npairs)
    def _(i):
        off0 = base + (2 * i) * WS
        off1 = off0 + WS
        pltpu.sync_copy(src_hbm.at[pl.ds(off0, WS)], sidx0)
        pltpu.sync_copy(dst_hbm.at[pl.ds(off0, WS)], didx0)
        gd0 = pltpu.async_copy(z_c.at[sidx0], rows0, g0)
        pltpu.sync_copy(src_hbm.at[pl.ds(off1, WS)], sidx1)
        pltpu.sync_copy(dst_hbm.at[pl.ds(off1, WS)], didx1)
        gd1 = pltpu.async_copy(z_c.at[sidx1], rows1, g1)
        gd0.wait()
        sd0 = pltpu.async_copy(rows0, acc.at[didx0], t0, add=True)
        gd1.wait()
        sd1 = pltpu.async_copy(rows1, acc.at[didx1], t1, add=True)
        sd0.wait()
        sd1.wait()

    offt = base + 2 * npairs * WS
    pltpu.sync_copy(src_hbm.at[pl.ds(offt, WS)], sidx0)
    pltpu.sync_copy(dst_hbm.at[pl.ds(offt, WS)], didx0)
    pltpu.sync_copy(z_c.at[sidx0], rows0)
    pltpu.sync_copy(rows0, acc.at[didx0], add=True)

    plsc.subcore_barrier()
    out_c = out_hbm.at[c]
    for off, sz in _CHUNKS:
        sl = pl.ds(s * SLN + off, sz)
        cda = pltpu.async_copy(acc.at[sl], rows0.at[pl.ds(0, sz)], g0)
        cdb = pltpu.async_copy(d2x_hbm.at[sl], rows1.at[pl.ds(0, sz)], g1)
        cda.wait()
        cdb.wait()

        @pl.loop(0, sz)
        def _(r):
            for hh in (0, 16):
                rows0[r, pl.ds(hh, 16)] = (rows0[r, pl.ds(hh, 16)]
                                           * rows1[r, pl.ds(hh, 16)])
        pltpu.sync_copy(rows0.at[pl.ds(0, sz)], out_c.at[sl])


def _dis_body(p_ref, dis_ref, sq_ref):
    deg = p_ref[...]
    dis = jnp.where(deg > 0, lax.rsqrt(deg), 0.0)
    dis_ref[...] = dis
    sq_ref[...] = deg * dis


def _scale1_body(d_ref, x_ref, o_ref):
    d = d_ref[...]
    o_ref[0, :, :] = d * x_ref[:, :H]
    o_ref[1, :, :] = d * x_ref[:, H:]


def _final_body(q_ref, e_ref, z1_ref, z2_ref, z3_ref, o_ref):
    zsum = z1_ref[...] + z2_ref[...] + z3_ref[...]
    cat = jnp.concatenate([zsum[0], zsum[1]], axis=1)
    o_ref[...] = 0.25 * (e_ref[...] + q_ref[...] * cat)


_BR = 5000
_COL = pl.BlockSpec((_BR, 1), lambda i: (i, 0))
_MAT = pl.BlockSpec((_BR, D), lambda i: (i, 0))
_HLF = pl.BlockSpec((NC, _BR, H), lambda i: (0, i, 0))

_tc_dis = pl.pallas_call(
    _dis_body,
    in_specs=[pl.BlockSpec((1, NP), lambda: (0, 0))],
    out_specs=(pl.BlockSpec((1, NP), lambda: (0, 0)),
               pl.BlockSpec((1, NP), lambda: (0, 0))),
    out_shape=(jax.ShapeDtypeStruct((1, NP), F32),
               jax.ShapeDtypeStruct((1, NP), F32)),
)
_tc_scale1 = pl.pallas_call(
    _scale1_body,
    grid=(N // _BR,),
    in_specs=[_COL, _MAT],
    out_specs=_HLF,
    out_shape=jax.ShapeDtypeStruct((NC, N, H), F32),
)
_tc_final = pl.pallas_call(
    _final_body,
    grid=(N // _BR,),
    in_specs=[_COL, _MAT, _HLF, _HLF, _HLF],
    out_specs=_MAT,
    out_shape=jax.ShapeDtypeStruct((N, D), F32),
)


def kernel(edge_index, emb):
    src = edge_index[0].astype(I32)
    dst = edge_index[1].astype(I32)

    ones_w = jnp.ones((WD,), F32)
    zdeg = jnp.zeros((SLD,), F32)
    zrow = jnp.zeros((WS, H), F32)

    deg, d2x = _sc_degree_kernel()(dst, ones_w, zdeg)
    dis, sq = _tc_dis(deg.reshape(1, NP))
    dis_col = dis[0, :N].reshape(N, 1)
    sq_col = sq[0, :N].reshape(N, 1)

    sc_scatter = _sc_scatter_kernel()
    z0 = _tc_scale1(dis_col, emb)
    z1 = sc_scatter(z0, src, dst, zrow, d2x)
    z2 = sc_scatter(z1, src, dst, zrow, d2x)
    z3 = sc_scatter(z2, src, dst, zrow, d2x)

    return _tc_final(sq_col, emb, z1, z2, z3)
